# baseline (speedup 1.0000x reference)
import functools

import jax
import jax.numpy as jnp
from jax import lax
from jax.experimental import pallas as pl
from jax.experimental.pallas import tpu as pltpu
from jax.experimental.pallas import tpu_sc as plsc

N_TOKENS = 16384
EMBED_DIM = 64
MAX_ROWS = 100000
HALF_ROWS = 50176
_LANES = 16
_DB = EMBED_DIM // _LANES
_IDX_CHUNK = 128
_HALF = 128
_TR_BLOCK = 7168


def _hsum(x):
    lanes = lax.iota(jnp.int32, _LANES)
    for k in (8, 4, 2, 1):
        perm = lax.bitwise_xor(lanes, jnp.int32(k))
        x = x + x.at[perm].get(mode="promise_in_bounds")
    return x


def _rsqrt_newton(x):
    i = lax.bitcast_convert_type(x, jnp.int32)
    i = jnp.int32(0x5F3759DF) - lax.shift_right_arithmetic(i, 1)
    y = lax.bitcast_convert_type(i, jnp.float32)
    for _ in range(1):
        y = y * (1.5 - 0.5 * x * y * y)
    return y


def _prep_body(xa0, xb0, e0, xa1, xb1, e1, xa2, xb2, e2, o0, o1, o2):
    dn = (((0,), (0,)), ((), ()))
    for xa, xb, e, o in ((xa0, xb0, e0, o0), (xa1, xb1, e1, o1),
                         (xa2, xb2, e2, o2)):
        ya = lax.dot_general(xa[...], e[...], dn,
                             preferred_element_type=jnp.float32)
        yb = lax.dot_general(xb[...], e[...], dn,
                             preferred_element_type=jnp.float32)
        o[:, 0:EMBED_DIM] = ya
        o[:, EMBED_DIM:2 * EMBED_DIM] = yb


@functools.lru_cache(maxsize=None)
def _build_tc_prep():
    grid = HALF_ROWS // _TR_BLOCK
    a_spec = pl.BlockSpec((EMBED_DIM, _TR_BLOCK), lambda i: (0, i))
    b_spec = pl.BlockSpec((EMBED_DIM, _TR_BLOCK),
                          lambda i: (0, i + HALF_ROWS // _TR_BLOCK))
    e_spec = pl.BlockSpec((EMBED_DIM, EMBED_DIM), lambda i: (0, 0))
    o_spec = pl.BlockSpec((_TR_BLOCK, 2 * EMBED_DIM), lambda i: (i, 0))
    o_type = jax.ShapeDtypeStruct((HALF_ROWS, 2 * EMBED_DIM), jnp.float32)
    return pl.pallas_call(
        _prep_body,
        grid=(grid,),
        in_specs=[a_spec, b_spec, e_spec] * 3,
        out_specs=(o_spec, o_spec, o_spec),
        out_shape=(o_type, o_type, o_type),
    )


_LN_BLOCK = 1024


def _ln_body(x_ref, tyv_ref, te_ref, par_ref, se_ref, so_ref, out_ref):
    x = x_ref[...]
    m = tyv_ref[...]
    par = par_ref[...]
    tew = te_ref[...]
    iota5 = lax.broadcasted_iota(jnp.int32, (1, 5), 1).astype(jnp.float32)
    acc = None
    for h, (sl, s_r) in enumerate(
            ((slice(0, EMBED_DIM), se_ref),
             (slice(EMBED_DIM, 2 * EMBED_DIM), so_ref))):
        ty = m[:, 2 * h:2 * h + 1]
        tv = m[:, 2 * h + 1:2 * h + 2]
        onehot = (ty == iota5).astype(jnp.float32)
        te = jnp.dot(onehot, tew, preferred_element_type=jnp.float32)
        xx = x[:, sl] + te + tv * par[0:1] + par[1:2]
        mu = jnp.mean(xx, axis=1, keepdims=True)
        d = xx - mu
        var = jnp.mean(d * d, axis=1, keepdims=True)
        y = d * lax.rsqrt(var + 1e-5) * par[2:3] + par[3:4]
        z = lax.dot_general(y, s_r[...], (((0,), (0,)), ((), ())),
                            preferred_element_type=jnp.float32)
        acc = z if acc is None else acc + z
    out_ref[...] = acc


@functools.lru_cache(maxsize=None)
def _build_tc_post():
    b2 = _LN_BLOCK // 2
    grid_i = (N_TOKENS // 2) // b2
    return pl.pallas_call(
        _ln_body,
        grid=(grid_i,),
        in_specs=[
            pl.BlockSpec((b2, 2 * EMBED_DIM), lambda i: (i, 0)),
            pl.BlockSpec((b2, 8), lambda i: (i, 0)),
            pl.BlockSpec((5, EMBED_DIM), lambda i: (0, 0)),
            pl.BlockSpec((8, EMBED_DIM), lambda i: (0, 0)),
            pl.BlockSpec((b2, _LN_BLOCK), lambda i: (0, 0)),
            pl.BlockSpec((b2, _LN_BLOCK), lambda i: (0, 0)),
        ],
        out_specs=pl.BlockSpec((EMBED_DIM, _LN_BLOCK), lambda i: (0, i)),
        out_shape=jax.ShapeDtypeStruct((EMBED_DIM, N_TOKENS), jnp.float32),
    )


@functools.lru_cache(maxsize=None)
def _build_sc_kernel():
    info = plsc.get_sparse_core_info()
    nc, ns = info.num_cores, info.num_subcores
    nw = nc * ns
    bpw = N_TOKENS // nw
    n_chunks = bpw // _IDX_CHUNK
    n_pass = bpw // _HALF
    gpp = _HALF // _LANES
    mesh = plsc.VectorSubcoreMesh(core_axis_name="c", subcore_axis_name="s")

    @functools.partial(
        pl.kernel,
        mesh=mesh,
        compiler_params=pltpu.CompilerParams(use_tc_tiling_on_sc=False),
        out_type=jax.ShapeDtypeStruct((N_TOKENS // 2, 2 * EMBED_DIM),
                                      jnp.float32),
        scratch_types=[
            pltpu.VMEM((n_chunks, _IDX_CHUNK), jnp.int32),
            pltpu.VMEM((n_chunks, _IDX_CHUNK), jnp.int32),
            pltpu.VMEM((n_chunks, _IDX_CHUNK), jnp.int32),
            pltpu.VMEM((bpw // _LANES, _LANES), jnp.int32),
            pltpu.VMEM((bpw // _LANES, _LANES), jnp.int32),
            pltpu.VMEM((bpw // _LANES, _LANES), jnp.int32),
            pltpu.VMEM((2, _HALF, 2 * EMBED_DIM), jnp.float32),
            pltpu.VMEM((2, _HALF, 2 * EMBED_DIM), jnp.float32),
            pltpu.VMEM((2, _HALF, 2 * EMBED_DIM), jnp.float32),
            pltpu.VMEM((_HALF // 2, 2 * EMBED_DIM), jnp.float32),
            pltpu.SemaphoreType.DMA,
            pltpu.SemaphoreType.DMA,
        ],
    )
    def sc_kernel(nidx_hbm, i1_hbm, i2_hbm,
                  ntab_hbm, t1_hbm, t2_hbm, out_hbm,
                  nidx_v, i1_v, i2_v, noffv, o1v, o2v,
                  rows_n, rows_1, rows_2, out_v, sem0, sem1):
        wid = lax.axis_index("s") * nc + lax.axis_index("c")
        cbase = wid * n_chunks

        pltpu.sync_copy(nidx_hbm.at[pl.ds(cbase, n_chunks)], nidx_v)
        pltpu.sync_copy(i1_hbm.at[pl.ds(cbase, n_chunks)], i1_v)
        pltpu.sync_copy(i2_hbm.at[pl.ds(cbase, n_chunks)], i2_v)

        half = jnp.full((_LANES,), HALF_ROWS, jnp.int32)
        z16 = jnp.zeros((_LANES,), jnp.int32)
        s16 = jnp.full((_LANES,), 64, jnp.int32)
        for idxv, offv in ((nidx_v, noffv), (i1_v, o1v), (i2_v, o2v)):
            for j in range(n_chunks):
                for k in range(_IDX_CHUNK // _LANES):
                    sl = pl.ds(k * _LANES, _LANES)
                    v = idxv[j, sl]
                    ge = v >= half
                    idxv[j, sl] = jnp.where(ge, v - half, v)
                    offv[j * (_IDX_CHUNK // _LANES) + k] = jnp.where(ge, s16, z16)
        sems = (sem0, sem1)

        def fire(p):
            b = p % 2
            return [
                pltpu.async_copy(ntab_hbm.at[nidx_v.at[p]], rows_n.at[b], sems[b]),
                pltpu.async_copy(t1_hbm.at[i1_v.at[p]], rows_1.at[b], sems[b]),
                pltpu.async_copy(t2_hbm.at[i2_v.at[p]], rows_2.at[b], sems[b]),
            ]

        pend = fire(0)
        for p in range(n_pass):
            for h in pend:
                h.wait()
            if p + 1 < n_pass:
                pend = fire(p + 1)
            b = p % 2

            def body(g, carry):
                gg = p * gpp + g
                on16 = noffv[gg]
                o116 = o1v[gg]
                o216 = o2v[gg]
                for l in range(_LANES):
                    t = g * _LANES + l
                    on = on16[l]
                    o1 = o116[l]
                    o2 = o216[l]
                    orow = g * (_LANES // 2) + l // 2
                    for db in range(_DB):
                        acc = (rows_n[b, t, pl.ds(on + db * _LANES, _LANES)]
                               + rows_1[b, t, pl.ds(o1 + db * _LANES, _LANES)]
                               + rows_2[b, t, pl.ds(o2 + db * _LANES, _LANES)])
                        col = (l % 2) * EMBED_DIM + db * _LANES
                        out_v[orow, pl.ds(col, _LANES)] = acc
                return carry

            lax.fori_loop(0, gpp, body, jnp.int32(0))
            prow = wid * (bpw // 2) + p * (_HALF // 2)
            pltpu.sync_copy(out_v, out_hbm.at[pl.ds(prow, _HALF // 2)])

    return sc_kernel


def kernel(token_types, token_values, node_indices, input1_indices, input2_indices,
           token_emb, value_W, value_b, node_idx_emb, input1_emb, input2_emb,
           combination_weights, ln_gamma, ln_beta):
    sc_kernel = _build_sc_kernel()
    tc_prep = _build_tc_prep()
    tc_post = _build_tc_post()
    cw = combination_weights
    te_w = token_emb * cw[0][None, :]
    vW2 = value_W[:, 0] * cw[1]
    vb2 = value_b * cw[1]
    params = jnp.concatenate([
        jnp.stack([vW2, vb2, ln_gamma, ln_beta]),
        jnp.zeros((4, EMBED_DIM), jnp.float32)], axis=0)
    nT = jnp.swapaxes(node_idx_emb, 0, 1)
    t1T = jnp.swapaxes(input1_emb, 0, 1)
    t2T = jnp.swapaxes(input2_emb, 0, 1)
    ntab, t1, t2 = tc_prep(nT, nT, jnp.diag(cw[2]),
                           t1T, t1T, jnp.diag(cw[3]),
                           t2T, t2T, jnp.diag(cw[4]))
    ni = node_indices.astype(jnp.int32)
    x1 = input1_indices.astype(jnp.int32)
    x2 = input2_indices.astype(jnp.int32)
    half = jnp.int32(HALF_ROWS)
    nidx = ni.reshape(-1, _IDX_CHUNK)
    i1 = x1.reshape(-1, _IDX_CHUNK)
    i2 = x2.reshape(-1, _IDX_CHUNK)
    combined = sc_kernel(nidx, i1, i2, ntab, t1, t2)
    tt = token_types.astype(jnp.int32)
    tv = token_values[:, 0]
    ttf = tt.astype(jnp.float32)
    tyv = jnp.stack([ttf[0::2], tv[0::2], ttf[1::2], tv[1::2],
                     ttf[0::2], tv[0::2], ttf[1::2], tv[1::2]],
                    axis=1)
    b2 = _LN_BLOCK // 2
    cols = jnp.arange(_LN_BLOCK)[None, :]
    rows = jnp.arange(b2)[:, None]
    se = (cols == 2 * rows).astype(jnp.float32)
    so = (cols == 2 * rows + 1).astype(jnp.float32)
    y = tc_post(combined, tyv, te_w, params, se, so)
    return y.T


# speedup vs baseline: 1.0506x; 1.0506x over previous
import functools

import jax
import jax.numpy as jnp
from jax import lax
from jax.experimental import pallas as pl
from jax.experimental.pallas import tpu as pltpu
from jax.experimental.pallas import tpu_sc as plsc

N_TOKENS = 16384
EMBED_DIM = 64
MAX_ROWS = 100000
HALF_ROWS = 50176
_LANES = 16
_DB = EMBED_DIM // _LANES
_IDX_CHUNK = 128
_HALF = 128
_TR_BLOCK = 3584


def _hsum(x):
    lanes = lax.iota(jnp.int32, _LANES)
    for k in (8, 4, 2, 1):
        perm = lax.bitwise_xor(lanes, jnp.int32(k))
        x = x + x.at[perm].get(mode="promise_in_bounds")
    return x


def _rsqrt_newton(x):
    i = lax.bitcast_convert_type(x, jnp.int32)
    i = jnp.int32(0x5F3759DF) - lax.shift_right_arithmetic(i, 1)
    y = lax.bitcast_convert_type(i, jnp.float32)
    for _ in range(1):
        y = y * (1.5 - 0.5 * x * y * y)
    return y


def _prep_body(xa0, xb0, e0, xa1, xb1, e1, xa2, xb2, e2, o0, o1, o2):
    dn = (((0,), (0,)), ((), ()))
    for xa, xb, e, o in ((xa0, xb0, e0, o0), (xa1, xb1, e1, o1),
                         (xa2, xb2, e2, o2)):
        ya = lax.dot_general(xa[...], e[...], dn,
                             preferred_element_type=jnp.float32)
        yb = lax.dot_general(xb[...], e[...], dn,
                             preferred_element_type=jnp.float32)
        o[:, 0:EMBED_DIM] = ya
        o[:, EMBED_DIM:2 * EMBED_DIM] = yb


@functools.lru_cache(maxsize=None)
def _build_tc_prep():
    grid = HALF_ROWS // _TR_BLOCK
    a_spec = pl.BlockSpec((EMBED_DIM, _TR_BLOCK), lambda i: (0, i))
    b_spec = pl.BlockSpec((EMBED_DIM, _TR_BLOCK),
                          lambda i: (0, i + HALF_ROWS // _TR_BLOCK))
    e_spec = pl.BlockSpec((EMBED_DIM, EMBED_DIM), lambda i: (0, 0))
    o_spec = pl.BlockSpec((_TR_BLOCK, 2 * EMBED_DIM), lambda i: (i, 0))
    o_type = jax.ShapeDtypeStruct((HALF_ROWS, 2 * EMBED_DIM), jnp.float32)
    return pl.pallas_call(
        _prep_body,
        grid=(grid,),
        in_specs=[a_spec, b_spec, e_spec] * 3,
        out_specs=(o_spec, o_spec, o_spec),
        out_shape=(o_type, o_type, o_type),
    )


_LN_BLOCK = 4096


def _ln_body(x_ref, tyv_ref, te_ref, par_ref, out_ref):
    x = x_ref[...]
    m = tyv_ref[...]
    par = par_ref[...]
    tew = te_ref[...]
    iota5 = lax.broadcasted_iota(jnp.int32, (1, 5), 1).astype(jnp.float32)
    for h, sl in enumerate((slice(0, EMBED_DIM),
                            slice(EMBED_DIM, 2 * EMBED_DIM))):
        ty = m[:, 2 * h:2 * h + 1]
        tv = m[:, 2 * h + 1:2 * h + 2]
        onehot = (ty == iota5).astype(jnp.float32)
        te = jnp.dot(onehot, tew, preferred_element_type=jnp.float32)
        xx = x[:, sl] + te + tv * par[0:1] + par[1:2]
        mu = jnp.mean(xx, axis=1, keepdims=True)
        d = xx - mu
        var = jnp.mean(d * d, axis=1, keepdims=True)
        out_ref[:, sl] = d * lax.rsqrt(var + 1e-5) * par[2:3] + par[3:4]


@functools.lru_cache(maxsize=None)
def _build_tc_post():
    b2 = _LN_BLOCK // 2
    grid_i = (N_TOKENS // 2) // b2
    return pl.pallas_call(
        _ln_body,
        grid=(grid_i,),
        in_specs=[
            pl.BlockSpec((b2, 2 * EMBED_DIM), lambda i: (i, 0)),
            pl.BlockSpec((b2, 8), lambda i: (i, 0)),
            pl.BlockSpec((5, EMBED_DIM), lambda i: (0, 0)),
            pl.BlockSpec((8, EMBED_DIM), lambda i: (0, 0)),
        ],
        out_specs=pl.BlockSpec((b2, 2 * EMBED_DIM), lambda i: (i, 0)),
        out_shape=jax.ShapeDtypeStruct((N_TOKENS // 2, 2 * EMBED_DIM),
                                       jnp.float32),
    )


@functools.lru_cache(maxsize=None)
def _build_sc_kernel():
    info = plsc.get_sparse_core_info()
    nc, ns = info.num_cores, info.num_subcores
    nw = nc * ns
    bpw = N_TOKENS // nw
    n_chunks = bpw // _IDX_CHUNK
    n_pass = bpw // _HALF
    gpp = _HALF // _LANES
    mesh = plsc.VectorSubcoreMesh(core_axis_name="c", subcore_axis_name="s")

    @functools.partial(
        pl.kernel,
        mesh=mesh,
        compiler_params=pltpu.CompilerParams(use_tc_tiling_on_sc=False),
        out_type=jax.ShapeDtypeStruct((N_TOKENS // 2, 2 * EMBED_DIM),
                                      jnp.float32),
        scratch_types=[
            pltpu.VMEM((n_chunks, _IDX_CHUNK), jnp.int32),
            pltpu.VMEM((n_chunks, _IDX_CHUNK), jnp.int32),
            pltpu.VMEM((n_chunks, _IDX_CHUNK), jnp.int32),
            pltpu.VMEM((bpw // _LANES, _LANES), jnp.int32),
            pltpu.VMEM((bpw // _LANES, _LANES), jnp.int32),
            pltpu.VMEM((bpw // _LANES, _LANES), jnp.int32),
            pltpu.VMEM((2, _HALF, 2 * EMBED_DIM), jnp.float32),
            pltpu.VMEM((2, _HALF, 2 * EMBED_DIM), jnp.float32),
            pltpu.VMEM((2, _HALF, 2 * EMBED_DIM), jnp.float32),
            pltpu.VMEM((_HALF // 2, 2 * EMBED_DIM), jnp.float32),
            pltpu.SemaphoreType.DMA,
            pltpu.SemaphoreType.DMA,
        ],
    )
    def sc_kernel(nidx_hbm, i1_hbm, i2_hbm,
                  ntab_hbm, t1_hbm, t2_hbm, out_hbm,
                  nidx_v, i1_v, i2_v, noffv, o1v, o2v,
                  rows_n, rows_1, rows_2, out_v, sem0, sem1):
        wid = lax.axis_index("s") * nc + lax.axis_index("c")
        cbase = wid * n_chunks

        pltpu.sync_copy(nidx_hbm.at[pl.ds(cbase, n_chunks)], nidx_v)
        pltpu.sync_copy(i1_hbm.at[pl.ds(cbase, n_chunks)], i1_v)
        pltpu.sync_copy(i2_hbm.at[pl.ds(cbase, n_chunks)], i2_v)

        half = jnp.full((_LANES,), HALF_ROWS, jnp.int32)
        z16 = jnp.zeros((_LANES,), jnp.int32)
        s16 = jnp.full((_LANES,), 64, jnp.int32)
        for idxv, offv in ((nidx_v, noffv), (i1_v, o1v), (i2_v, o2v)):
            for j in range(n_chunks):
                for k in range(_IDX_CHUNK // _LANES):
                    sl = pl.ds(k * _LANES, _LANES)
                    v = idxv[j, sl]
                    ge = v >= half
                    idxv[j, sl] = jnp.where(ge, v - half, v)
                    offv[j * (_IDX_CHUNK // _LANES) + k] = jnp.where(ge, s16, z16)
        sems = (sem0, sem1)

        def fire(p):
            b = p % 2
            return [
                pltpu.async_copy(ntab_hbm.at[nidx_v.at[p]], rows_n.at[b], sems[b]),
                pltpu.async_copy(t1_hbm.at[i1_v.at[p]], rows_1.at[b], sems[b]),
                pltpu.async_copy(t2_hbm.at[i2_v.at[p]], rows_2.at[b], sems[b]),
            ]

        pend = fire(0)
        for p in range(n_pass):
            for h in pend:
                h.wait()
            if p + 1 < n_pass:
                pend = fire(p + 1)
            b = p % 2

            def body(g, carry):
                gg = p * gpp + g
                on16 = noffv[gg]
                o116 = o1v[gg]
                o216 = o2v[gg]
                for l in range(_LANES):
                    t = g * _LANES + l
                    on = on16[l]
                    o1 = o116[l]
                    o2 = o216[l]
                    orow = g * (_LANES // 2) + l // 2
                    for db in range(_DB):
                        acc = (rows_n[b, t, pl.ds(on + db * _LANES, _LANES)]
                               + rows_1[b, t, pl.ds(o1 + db * _LANES, _LANES)]
                               + rows_2[b, t, pl.ds(o2 + db * _LANES, _LANES)])
                        col = (l % 2) * EMBED_DIM + db * _LANES
                        out_v[orow, pl.ds(col, _LANES)] = acc
                return carry

            lax.fori_loop(0, gpp, body, jnp.int32(0))
            prow = wid * (bpw // 2) + p * (_HALF // 2)
            pltpu.sync_copy(out_v, out_hbm.at[pl.ds(prow, _HALF // 2)])

    return sc_kernel


def kernel(token_types, token_values, node_indices, input1_indices, input2_indices,
           token_emb, value_W, value_b, node_idx_emb, input1_emb, input2_emb,
           combination_weights, ln_gamma, ln_beta):
    sc_kernel = _build_sc_kernel()
    tc_prep = _build_tc_prep()
    tc_post = _build_tc_post()
    cw = combination_weights
    te_w = token_emb * cw[0][None, :]
    vW2 = value_W[:, 0] * cw[1]
    vb2 = value_b * cw[1]
    params = jnp.concatenate([
        jnp.stack([vW2, vb2, ln_gamma, ln_beta]),
        jnp.zeros((4, EMBED_DIM), jnp.float32)], axis=0)
    nT = jnp.swapaxes(node_idx_emb, 0, 1)
    t1T = jnp.swapaxes(input1_emb, 0, 1)
    t2T = jnp.swapaxes(input2_emb, 0, 1)
    ntab, t1, t2 = tc_prep(nT, nT, jnp.diag(cw[2]),
                           t1T, t1T, jnp.diag(cw[3]),
                           t2T, t2T, jnp.diag(cw[4]))
    ni = node_indices.astype(jnp.int32)
    x1 = input1_indices.astype(jnp.int32)
    x2 = input2_indices.astype(jnp.int32)
    half = jnp.int32(HALF_ROWS)
    nidx = ni.reshape(-1, _IDX_CHUNK)
    i1 = x1.reshape(-1, _IDX_CHUNK)
    i2 = x2.reshape(-1, _IDX_CHUNK)
    combined = sc_kernel(nidx, i1, i2, ntab, t1, t2)
    tt = token_types.astype(jnp.int32)
    tv = token_values[:, 0]
    ttf = tt.astype(jnp.float32)
    tyv = jnp.stack([ttf[0::2], tv[0::2], ttf[1::2], tv[1::2],
                     ttf[0::2], tv[0::2], ttf[1::2], tv[1::2]],
                    axis=1)
    y = tc_post(combined, tyv, te_w, params)
    return y.reshape(N_TOKENS, EMBED_DIM)
